# Initial kernel scaffold; baseline (speedup 1.0000x reference)
import jax
import jax.numpy as jnp
from jax.experimental import pallas as pl


def kernel(x, norm, e_h, bases0, w_comp0, loop_w0, h_bias0, bases1, w_comp1, loop_w1, h_bias1, msg_loop_W, msg_loop_b, rel_W, rel_b, bias_v, edge_index, rel_type):
    raise NotImplementedError("write your pallas kernel here")


# speedup vs baseline: 14.9396x; 14.9396x over previous
import functools

import jax
import jax.numpy as jnp
from jax import lax
from jax.experimental import pallas as pl
from jax.experimental.pallas import tpu as pltpu
from jax.experimental.pallas import tpu_sc as plsc

_NTILES = 32
_BLKE = 128
_AGG_ROWS = 10240


def _transform(x, bases, w_comp, loop_w):
    n, d = x.shape
    r_, nb = w_comp.shape
    bl = 1000

    def body(x_ref, bases_ref, w_comp_ref, loop_w_ref, out_ref):
        cols = []
        for r in range(r_):
            w = w_comp_ref[r, 0] * bases_ref[0]
            for b in range(1, nb):
                w = w + w_comp_ref[r, b] * bases_ref[b]
            cols.append(w)
        cols.append(loop_w_ref[...])
        wcat = jnp.concatenate(cols, axis=1)
        out_ref[...] = jnp.dot(x_ref[...], wcat,
                               preferred_element_type=jnp.float32)

    return pl.pallas_call(
        body,
        grid=(n // bl,),
        in_specs=[
            pl.BlockSpec((bl, d), lambda i: (i, 0)),
            pl.BlockSpec((nb, d, d), lambda i: (0, 0, 0)),
            pl.BlockSpec(memory_space=pltpu.SMEM),
            pl.BlockSpec((d, d), lambda i: (0, 0)),
        ],
        out_specs=pl.BlockSpec((bl, (r_ + 1) * d), lambda i: (i, 0)),
        out_shape=jax.ShapeDtypeStruct((n, (r_ + 1) * d), jnp.float32),
    )(x, bases, w_comp, loop_w)


def _sc_agg(hflat, src_t, rel_t, dst_t, nblk, d):
    mesh = plsc.VectorSubcoreMesh(core_axis_name="c", subcore_axis_name="s")

    @functools.partial(
        pl.kernel,
        out_type=jax.ShapeDtypeStruct((2, _AGG_ROWS, d), jnp.float32),
        mesh=mesh,
        scratch_types=[
            pltpu.VMEM((nblk, _BLKE), jnp.int32),
            pltpu.VMEM((nblk, _BLKE), jnp.int32),
            pltpu.VMEM((_BLKE, d), jnp.float32),
            pltpu.VMEM_SHARED((_AGG_ROWS, d), jnp.float32),
            pltpu.SemaphoreType.DMA,
        ],
    )
    def k(hflat_hbm, src_hbm, rel_hbm, dst_hbm, out_hbm,
          flat_v, dst_v, rows_v, agg_sh, sem):
        c = lax.axis_index("c")
        s = lax.axis_index("s")
        wid = c * 16 + s

        def zbody(i, _):
            for kk in range(d // 16):
                rows_v[i, pl.ds(kk * 16, 16)] = jnp.zeros((16,), jnp.float32)
            return _
        lax.fori_loop(0, _BLKE, zbody, None)

        def zagg(j, _):
            pltpu.sync_copy(rows_v, agg_sh.at[pl.ds(s * 640 + j * _BLKE, _BLKE)])
            return _
        lax.fori_loop(0, 5, zagg, None)

        pltpu.sync_copy(src_hbm.at[wid], flat_v)
        pltpu.sync_copy(rel_hbm.at[wid], dst_v)

        def fbody(j, _):
            for kk in range(_BLKE // 16):
                sl = pl.ds(kk * 16, 16)
                flat_v[j, sl] = flat_v[j, sl] * 9 + dst_v[j, sl]
            return _
        lax.fori_loop(0, nblk, fbody, None)

        pltpu.sync_copy(dst_hbm.at[wid], dst_v)

        plsc.subcore_barrier()

        def mbody(j, _):
            pltpu.async_copy(hflat_hbm.at[flat_v.at[j]], rows_v, sem).wait()
            pltpu.sync_copy(rows_v, agg_sh.at[dst_v.at[j]], add=True)
            return _
        lax.fori_loop(0, nblk, mbody, None)

        plsc.subcore_barrier()

        def obody(j, _):
            r0 = s * 640 + j * _BLKE
            pltpu.sync_copy(agg_sh.at[pl.ds(r0, _BLKE)], rows_v)
            pltpu.sync_copy(rows_v, out_hbm.at[c].at[pl.ds(r0, _BLKE)])
            return _
        lax.fori_loop(0, 5, obody, None)

    return k(hflat, src_t, rel_t, dst_t)


def _combine(aggpair, hcat, norm, h_bias, r_):
    n, d = norm.shape[0], h_bias.shape[0]
    bl = 1000

    def body(agg_ref, self_ref, norm_ref, bias_ref, out_ref):
        a = agg_ref[0] + agg_ref[1]
        out_ref[...] = jnp.maximum(
            norm_ref[...] * a + self_ref[...] + bias_ref[...], 0.0)

    return pl.pallas_call(
        body,
        grid=(n // bl,),
        in_specs=[
            pl.BlockSpec((2, bl, d), lambda i: (0, i, 0)),
            pl.BlockSpec((bl, d), lambda i: (i, r_)),
            pl.BlockSpec((bl, 1), lambda i: (i, 0)),
            pl.BlockSpec((1, d), lambda i: (0, 0)),
        ],
        out_specs=pl.BlockSpec((bl, d), lambda i: (i, 0)),
        out_shape=jax.ShapeDtypeStruct((n, d), jnp.float32),
    )(aggpair, hcat, norm, h_bias.reshape(1, d))


def _final(aggpair, hcat, norm, h_bias, msg_w, msg_b, r_):
    n, d = norm.shape[0], h_bias.shape[0]
    bl = 1000

    def body(agg_ref, self_ref, norm_ref, bias_ref, w_ref, b_ref, out_ref):
        a = agg_ref[0] + agg_ref[1]
        h2 = jnp.maximum(
            norm_ref[...] * a + self_ref[...] + bias_ref[...], 0.0)
        out_ref[...] = jnp.maximum(
            jnp.dot(h2, w_ref[...], preferred_element_type=jnp.float32)
            + b_ref[...], 0.0)

    return pl.pallas_call(
        body,
        grid=(n // bl,),
        in_specs=[
            pl.BlockSpec((2, bl, d), lambda i: (0, i, 0)),
            pl.BlockSpec((bl, d), lambda i: (i, r_)),
            pl.BlockSpec((bl, 1), lambda i: (i, 0)),
            pl.BlockSpec((1, d), lambda i: (0, 0)),
            pl.BlockSpec((d, d), lambda i: (0, 0)),
            pl.BlockSpec((1, d), lambda i: (0, 0)),
        ],
        out_specs=pl.BlockSpec((bl, d), lambda i: (i, 0)),
        out_shape=jax.ShapeDtypeStruct((n, d), jnp.float32),
    )(aggpair, hcat, norm, h_bias.reshape(1, d), msg_w, msg_b.reshape(1, d))


def _edge(ehr, rel_w, rel_b):
    m = ehr.shape[0]
    de = rel_w.shape[0]
    g = 128 // de
    bl = 5000

    def body(x_ref, w_ref, b_ref, out_ref):
        ii = lax.broadcasted_iota(jnp.int32, (128, de), 0)
        jj = lax.broadcasted_iota(jnp.int32, (128, de), 1)
        p = (ii % de == jj).astype(jnp.float32)
        i2 = lax.broadcasted_iota(jnp.int32, (de, 128), 0)
        j2 = lax.broadcasted_iota(jnp.int32, (de, 128), 1)
        q = (j2 % de == i2).astype(jnp.float32)
        pw = jnp.dot(p, w_ref[...], preferred_element_type=jnp.float32)
        w8 = jnp.dot(pw, q, preferred_element_type=jnp.float32)
        bi = lax.broadcasted_iota(jnp.int32, (128, 128), 0)
        bj = lax.broadcasted_iota(jnp.int32, (128, 128), 1)
        w8 = jnp.where(bi // de == bj // de, w8, 0.0)
        b128 = jnp.dot(b_ref[...], q, preferred_element_type=jnp.float32)
        out_ref[...] = jnp.dot(x_ref[...], w8,
                               preferred_element_type=jnp.float32) + b128

    del g
    return pl.pallas_call(
        body,
        grid=(m // bl,),
        in_specs=[
            pl.BlockSpec((bl, 128), lambda i: (i, 0)),
            pl.BlockSpec((de, de), lambda i: (0, 0)),
            pl.BlockSpec((1, de), lambda i: (0, 0)),
        ],
        out_specs=pl.BlockSpec((bl, 128), lambda i: (i, 0)),
        out_shape=jax.ShapeDtypeStruct((m, 128), jnp.float32),
    )(ehr, rel_w, rel_b.reshape(1, de))


def kernel(x, norm, e_h, bases0, w_comp0, loop_w0, h_bias0,
           bases1, w_comp1, loop_w1, h_bias1, msg_loop_W, msg_loop_b,
           rel_W, rel_b, bias_v, edge_index, rel_type):
    n, d = x.shape
    e = edge_index.shape[1]
    r_ = w_comp0.shape[0]
    de = rel_W.shape[0]

    ept = _BLKE * -(-e // (_NTILES * _BLKE))
    epad = ept * _NTILES
    nblk = ept // _BLKE

    src = edge_index[0].astype(jnp.int32)
    dst = edge_index[1].astype(jnp.int32)
    rel = rel_type.astype(jnp.int32)
    pad = epad - e
    src_t = jnp.concatenate([src, jnp.zeros((pad,), jnp.int32)]) \
        .reshape(_NTILES, nblk, _BLKE)
    rel_t = jnp.concatenate([rel, jnp.zeros((pad,), jnp.int32)]) \
        .reshape(_NTILES, nblk, _BLKE)
    dst_t = jnp.concatenate([dst, jnp.full((pad,), n, jnp.int32)]) \
        .reshape(_NTILES, nblk, _BLKE)

    hcat0 = _transform(x, bases0, w_comp0, loop_w0)
    agg0 = _sc_agg(hcat0.reshape(n * (r_ + 1), d), src_t, rel_t, dst_t, nblk, d)
    h1 = _combine(agg0, hcat0, norm, h_bias0, r_)
    hcat1 = _transform(h1, bases1, w_comp1, loop_w1)
    agg1 = _sc_agg(hcat1.reshape(n * (r_ + 1), d), src_t, rel_t, dst_t, nblk, d)
    hh = _final(agg1, hcat1, norm, h_bias1, msg_loop_W, msg_loop_b, r_)
    eh = _edge(e_h.reshape(e * de // 128, 128), rel_W, rel_b).reshape(e, de)
    del bias_v
    return hh, eh


# speedup vs baseline: 24.6375x; 1.6491x over previous
import functools

import jax
import jax.numpy as jnp
from jax import lax
from jax.experimental import pallas as pl
from jax.experimental.pallas import tpu as pltpu
from jax.experimental.pallas import tpu_sc as plsc

_NTILES = 32
_BLKE = 80
_NBT = 125
_AGG_ROWS = 10112


def _transform(x, bases, w_comp, loop_w):
    n, d = x.shape
    r_, nb = w_comp.shape
    bl = 1000

    def body(x_ref, bases_ref, w_comp_ref, loop_w_ref, out_ref):
        cols = []
        for r in range(r_):
            w = w_comp_ref[r, 0] * bases_ref[0]
            for b in range(1, nb):
                w = w + w_comp_ref[r, b] * bases_ref[b]
            cols.append(w)
        cols.append(loop_w_ref[...])
        wcat = jnp.concatenate(cols, axis=1)
        out_ref[...] = jnp.dot(x_ref[...], wcat,
                               preferred_element_type=jnp.float32)

    return pl.pallas_call(
        body,
        grid=(n // bl,),
        in_specs=[
            pl.BlockSpec((bl, d), lambda i: (i, 0)),
            pl.BlockSpec((nb, d, d), lambda i: (0, 0, 0)),
            pl.BlockSpec(memory_space=pltpu.SMEM),
            pl.BlockSpec((d, d), lambda i: (0, 0)),
        ],
        out_specs=pl.BlockSpec((bl, (r_ + 1) * d), lambda i: (i, 0)),
        out_shape=jax.ShapeDtypeStruct((n, (r_ + 1) * d), jnp.float32),
    )(x, bases, w_comp, loop_w)


def _sc_agg(hflat, src_t, rel_t, dst_t, d):
    mesh = plsc.VectorSubcoreMesh(core_axis_name="c", subcore_axis_name="s")
    rpt = _AGG_ROWS // 16
    nfull = rpt // _BLKE
    tail = rpt - nfull * _BLKE

    @functools.partial(
        pl.kernel,
        out_type=jax.ShapeDtypeStruct((2, _AGG_ROWS, d), jnp.float32),
        mesh=mesh,
        scratch_types=[
            pltpu.VMEM((64, _BLKE), jnp.int32),
            pltpu.VMEM((64, _BLKE), jnp.int32),
            pltpu.VMEM((2, _BLKE, d), jnp.float32),
            pltpu.VMEM_SHARED((_AGG_ROWS, d), jnp.float32),
            pltpu.SemaphoreType.DMA,
            pltpu.SemaphoreType.DMA,
        ],
    )
    def k(hflat_hbm, src_hbm, rel_hbm, dst_hbm, out_hbm,
          flat_v, dst_v, rows_v, agg_sh, sem0, sem1):
        c = lax.axis_index("c")
        s = lax.axis_index("s")
        wid = c * 16 + s
        buf0 = rows_v.at[0]
        buf1 = rows_v.at[1]

        def zbody(i, _):
            for kk in range(d // 16):
                rows_v[0, i, pl.ds(kk * 16, 16)] = jnp.zeros((16,), jnp.float32)
            return _
        lax.fori_loop(0, _BLKE, zbody, None)
        for j in range(nfull):
            pltpu.sync_copy(buf0, agg_sh.at[pl.ds(s * rpt + j * _BLKE, _BLKE)])
        pltpu.sync_copy(buf0.at[pl.ds(0, tail)],
                        agg_sh.at[pl.ds(s * rpt + nfull * _BLKE, tail)])

        plsc.subcore_barrier()

        def gstart(j, buf, sem):
            pltpu.async_copy(hflat_hbm.at[flat_v.at[j]], buf, sem)

        def gwait(j, buf, sem):
            pltpu.make_async_copy(hflat_hbm.at[flat_v.at[j]], buf, sem).wait()

        def scat(j, buf):
            pltpu.sync_copy(buf, agg_sh.at[dst_v.at[j]], add=True)

        def run_edges(ofs, nb):
            pltpu.sync_copy(src_hbm.at[wid].at[pl.ds(ofs, nb)],
                            flat_v.at[pl.ds(0, nb)])
            pltpu.sync_copy(rel_hbm.at[wid].at[pl.ds(ofs, nb)],
                            dst_v.at[pl.ds(0, nb)])

            def fbody(j, _):
                for kk in range(_BLKE // 16):
                    sl = pl.ds(kk * 16, 16)
                    flat_v[j, sl] = flat_v[j, sl] * 9 + dst_v[j, sl]
                return _
            lax.fori_loop(0, nb, fbody, None)

            pltpu.sync_copy(dst_hbm.at[wid].at[pl.ds(ofs, nb)],
                            dst_v.at[pl.ds(0, nb)])

            def mbody(i, _):
                j = 2 * i
                gstart(j + 1, buf1, sem1)
                gwait(j, buf0, sem0)
                scat(j, buf0)
                gstart(j + 2, buf0, sem0)
                gwait(j + 1, buf1, sem1)
                scat(j + 1, buf1)
                return _

            gstart(0, buf0, sem0)
            if nb % 2:
                lax.fori_loop(0, (nb - 1) // 2, mbody, None)
                gwait(nb - 1, buf0, sem0)
                scat(nb - 1, buf0)
            else:
                lax.fori_loop(0, nb // 2 - 1, mbody, None)
                gstart(nb - 1, buf1, sem1)
                gwait(nb - 2, buf0, sem0)
                scat(nb - 2, buf0)
                gwait(nb - 1, buf1, sem1)
                scat(nb - 1, buf1)

        run_edges(0, 64)
        run_edges(64, _NBT - 64)

        plsc.subcore_barrier()

        for j in range(nfull):
            r0 = s * rpt + j * _BLKE
            pltpu.sync_copy(agg_sh.at[pl.ds(r0, _BLKE)], buf0)
            pltpu.sync_copy(buf0, out_hbm.at[c].at[pl.ds(r0, _BLKE)])
        r0t = s * rpt + nfull * _BLKE
        pltpu.sync_copy(agg_sh.at[pl.ds(r0t, tail)], buf0.at[pl.ds(0, tail)])
        pltpu.sync_copy(buf0.at[pl.ds(0, tail)], out_hbm.at[c].at[pl.ds(r0t, tail)])

    return k(hflat, src_t, rel_t, dst_t)


def _combine(aggpair, hcat, norm, h_bias, r_):
    n, d = norm.shape[0], h_bias.shape[0]
    bl = 1000

    def body(agg_ref, self_ref, norm_ref, bias_ref, out_ref):
        a = agg_ref[0] + agg_ref[1]
        out_ref[...] = jnp.maximum(
            norm_ref[...] * a + self_ref[...] + bias_ref[...], 0.0)

    return pl.pallas_call(
        body,
        grid=(n // bl,),
        in_specs=[
            pl.BlockSpec((2, bl, d), lambda i: (0, i, 0)),
            pl.BlockSpec((bl, d), lambda i: (i, r_)),
            pl.BlockSpec((bl, 1), lambda i: (i, 0)),
            pl.BlockSpec((1, d), lambda i: (0, 0)),
        ],
        out_specs=pl.BlockSpec((bl, d), lambda i: (i, 0)),
        out_shape=jax.ShapeDtypeStruct((n, d), jnp.float32),
    )(aggpair, hcat, norm, h_bias.reshape(1, d))


def _final(aggpair, hcat, norm, h_bias, msg_w, msg_b, r_):
    n, d = norm.shape[0], h_bias.shape[0]
    bl = 1000

    def body(agg_ref, self_ref, norm_ref, bias_ref, w_ref, b_ref, out_ref):
        a = agg_ref[0] + agg_ref[1]
        h2 = jnp.maximum(
            norm_ref[...] * a + self_ref[...] + bias_ref[...], 0.0)
        out_ref[...] = jnp.maximum(
            jnp.dot(h2, w_ref[...], preferred_element_type=jnp.float32)
            + b_ref[...], 0.0)

    return pl.pallas_call(
        body,
        grid=(n // bl,),
        in_specs=[
            pl.BlockSpec((2, bl, d), lambda i: (0, i, 0)),
            pl.BlockSpec((bl, d), lambda i: (i, r_)),
            pl.BlockSpec((bl, 1), lambda i: (i, 0)),
            pl.BlockSpec((1, d), lambda i: (0, 0)),
            pl.BlockSpec((d, d), lambda i: (0, 0)),
            pl.BlockSpec((1, d), lambda i: (0, 0)),
        ],
        out_specs=pl.BlockSpec((bl, d), lambda i: (i, 0)),
        out_shape=jax.ShapeDtypeStruct((n, d), jnp.float32),
    )(aggpair, hcat, norm, h_bias.reshape(1, d), msg_w, msg_b.reshape(1, d))


def _edge(ehr, rel_w, rel_b):
    m = ehr.shape[0]
    de = rel_w.shape[0]
    g = 128 // de
    bl = 5000

    def body(x_ref, w_ref, b_ref, out_ref):
        ii = lax.broadcasted_iota(jnp.int32, (128, de), 0)
        jj = lax.broadcasted_iota(jnp.int32, (128, de), 1)
        p = (ii % de == jj).astype(jnp.float32)
        i2 = lax.broadcasted_iota(jnp.int32, (de, 128), 0)
        j2 = lax.broadcasted_iota(jnp.int32, (de, 128), 1)
        q = (j2 % de == i2).astype(jnp.float32)
        pw = jnp.dot(p, w_ref[...], preferred_element_type=jnp.float32)
        w8 = jnp.dot(pw, q, preferred_element_type=jnp.float32)
        bi = lax.broadcasted_iota(jnp.int32, (128, 128), 0)
        bj = lax.broadcasted_iota(jnp.int32, (128, 128), 1)
        w8 = jnp.where(bi // de == bj // de, w8, 0.0)
        b128 = jnp.dot(b_ref[...], q, preferred_element_type=jnp.float32)
        out_ref[...] = jnp.dot(x_ref[...], w8,
                               preferred_element_type=jnp.float32) + b128

    del g
    return pl.pallas_call(
        body,
        grid=(m // bl,),
        in_specs=[
            pl.BlockSpec((bl, 128), lambda i: (i, 0)),
            pl.BlockSpec((de, de), lambda i: (0, 0)),
            pl.BlockSpec((1, de), lambda i: (0, 0)),
        ],
        out_specs=pl.BlockSpec((bl, 128), lambda i: (i, 0)),
        out_shape=jax.ShapeDtypeStruct((m, 128), jnp.float32),
    )(ehr, rel_w, rel_b.reshape(1, de))


def kernel(x, norm, e_h, bases0, w_comp0, loop_w0, h_bias0,
           bases1, w_comp1, loop_w1, h_bias1, msg_loop_W, msg_loop_b,
           rel_W, rel_b, bias_v, edge_index, rel_type):
    n, d = x.shape
    e = edge_index.shape[1]
    r_ = w_comp0.shape[0]
    de = rel_W.shape[0]

    src_t = edge_index[0].astype(jnp.int32).reshape(_NTILES, _NBT, _BLKE)
    dst_t = edge_index[1].astype(jnp.int32).reshape(_NTILES, _NBT, _BLKE)
    rel_t = rel_type.astype(jnp.int32).reshape(_NTILES, _NBT, _BLKE)

    hcat0 = _transform(x, bases0, w_comp0, loop_w0)
    agg0 = _sc_agg(hcat0.reshape(n * (r_ + 1), d), src_t, rel_t, dst_t, d)
    h1 = _combine(agg0, hcat0, norm, h_bias0, r_)
    hcat1 = _transform(h1, bases1, w_comp1, loop_w1)
    agg1 = _sc_agg(hcat1.reshape(n * (r_ + 1), d), src_t, rel_t, dst_t, d)
    hh = _final(agg1, hcat1, norm, h_bias1, msg_loop_W, msg_loop_b, r_)
    eh = _edge(e_h.reshape(e * de // 128, 128), rel_W, rel_b).reshape(e, de)
    del bias_v
    return hh, eh
